# Initial kernel scaffold; baseline (speedup 1.0000x reference)
import jax
import jax.numpy as jnp
from jax.experimental import pallas as pl


def kernel(X, edge_index, D, Wp1, bp1, Ws1, bs1, Wpe1, bpe1, Wse1, bse1, Wp2, bp2, Ws2, bs2, Wpe2, bpe2, Wse2, bse2, Wc, bc):
    raise NotImplementedError("write your pallas kernel here")


# speedup vs baseline: 1.8712x; 1.8712x over previous
import functools

import jax
import jax.numpy as jnp
from jax import lax
from jax.experimental import pallas as pl
from jax.experimental.pallas import tpu as pltpu
from jax.experimental.pallas import tpu_sc as plsc

N = 10000
E = 320000
DF = 128
H = 32

NC = 2
NS = 16
NW = NC * NS
EPW = E // NW
CH = 80
NCHUNK = EPW // CH
NPAD = 10240
NPT = NPAD // NS

_mesh = plsc.VectorSubcoreMesh(core_axis_name="c", subcore_axis_name="s")

_f32 = jnp.float32


@functools.partial(
    pl.kernel,
    out_type=jax.ShapeDtypeStruct((E, DF), _f32),
    mesh=_mesh,
    scratch_types=[
        pltpu.VMEM((CH,), jnp.int32),
        pltpu.VMEM((CH,), jnp.int32),
        pltpu.VMEM((CH, DF), _f32),
        pltpu.VMEM((CH, DF), _f32),
        pltpu.SemaphoreType.DMA,
    ],
)
def _k1_absdiff(x_hbm, src_hbm, dst_hbm, w0_hbm, is_v, id_v, xs_v, xd_v, sem):
    wid = lax.axis_index("s") * NC + lax.axis_index("c")

    def chunk(c, carry):
        base = pl.multiple_of(wid * EPW + c * CH, 8)
        pltpu.sync_copy(src_hbm.at[pl.ds(base, CH)], is_v)
        pltpu.sync_copy(dst_hbm.at[pl.ds(base, CH)], id_v)
        pltpu.async_copy(x_hbm.at[is_v], xs_v, sem).wait()
        pltpu.async_copy(x_hbm.at[id_v], xd_v, sem).wait()

        def row(r, carry2):
            for j in range(DF // 16):
                sl = pl.ds(j * 16, 16)
                xs_v[r, sl] = jnp.abs(xs_v[r, sl] - xd_v[r, sl])
            return carry2

        lax.fori_loop(0, CH, row, 0)
        pltpu.sync_copy(xs_v, w0_hbm.at[pl.ds(base, CH)])
        return carry

    lax.fori_loop(0, NCHUNK, chunk, 0)


@functools.partial(
    pl.kernel,
    out_type=jax.ShapeDtypeStruct((NC, NPAD, H), _f32),
    mesh=_mesh,
    scratch_types=[
        pltpu.VMEM((CH,), jnp.int32),
        pltpu.VMEM((CH, H), _f32),
        pltpu.VMEM((NPT, H), _f32),
        pltpu.VMEM_SHARED((NPAD, H), _f32),
        pltpu.SemaphoreType.DMA,
    ],
    compiler_params=pltpu.CompilerParams(use_tc_tiling_on_sc=False),
)
def _k3_segsum(a0_hbm, src_hbm, out_hbm, idx_v, val_v, z_v, agg_sh, sem):
    cid = lax.axis_index("c")
    sid = lax.axis_index("s")
    wid = sid * NC + cid

    def zrow(r, carry):
        for j in range(H // 16):
            z_v[r, pl.ds(j * 16, 16)] = jnp.zeros((16,), _f32)
        return carry

    lax.fori_loop(0, NPT, zrow, 0)
    pltpu.sync_copy(z_v, agg_sh.at[pl.ds(sid * NPT, NPT)])
    plsc.subcore_barrier()

    def chunk(c, carry):
        base = pl.multiple_of(wid * EPW + c * CH, 8)
        pltpu.sync_copy(src_hbm.at[pl.ds(base, CH)], idx_v)
        pltpu.sync_copy(a0_hbm.at[pl.ds(base, CH)], val_v)
        pltpu.sync_copy(val_v, agg_sh.at[idx_v], add=True)
        return carry

    lax.fori_loop(0, NCHUNK, chunk, 0)
    plsc.subcore_barrier()
    pltpu.sync_copy(agg_sh.at[pl.ds(sid * NPT, NPT)], z_v)
    pltpu.sync_copy(z_v, out_hbm.at[cid, pl.ds(sid * NPT, NPT)])


@functools.partial(
    pl.kernel,
    out_type=[
        jax.ShapeDtypeStruct((E, H), _f32),
        jax.ShapeDtypeStruct((NC, NPAD, H), _f32),
    ],
    mesh=_mesh,
    scratch_types=[
        pltpu.VMEM((CH,), jnp.int32),
        pltpu.VMEM((CH,), jnp.int32),
        pltpu.VMEM((CH, H), _f32),
        pltpu.VMEM((CH, H), _f32),
        pltpu.VMEM((CH, H), _f32),
        pltpu.VMEM((NPT, H), _f32),
        pltpu.VMEM_SHARED((NPAD, H), _f32),
        pltpu.SemaphoreType.DMA,
    ],
    compiler_params=pltpu.CompilerParams(use_tc_tiling_on_sc=False),
)
def _k5_edge1(y1_hbm, y2_hbm, b0_hbm, src_hbm, dst_hbm, w1_hbm, agg_hbm,
              is_v, id_v, y1_v, y2_v, b0_v, z_v, agg_sh, sem):
    cid = lax.axis_index("c")
    sid = lax.axis_index("s")
    wid = sid * NC + cid

    def zrow(r, carry):
        for j in range(H // 16):
            z_v[r, pl.ds(j * 16, 16)] = jnp.zeros((16,), _f32)
        return carry

    lax.fori_loop(0, NPT, zrow, 0)
    pltpu.sync_copy(z_v, agg_sh.at[pl.ds(sid * NPT, NPT)])
    plsc.subcore_barrier()

    def chunk(c, carry):
        base = pl.multiple_of(wid * EPW + c * CH, 8)
        pltpu.sync_copy(src_hbm.at[pl.ds(base, CH)], is_v)
        pltpu.sync_copy(dst_hbm.at[pl.ds(base, CH)], id_v)
        pltpu.async_copy(y1_hbm.at[is_v], y1_v, sem).wait()
        pltpu.async_copy(y2_hbm.at[id_v], y2_v, sem).wait()
        pltpu.sync_copy(b0_hbm.at[pl.ds(base, CH)], b0_v)

        def row(r, carry2):
            for j in range(H // 16):
                sl = pl.ds(j * 16, 16)
                b0_v[r, sl] = jnp.maximum(
                    y1_v[r, sl] + y2_v[r, sl] + b0_v[r, sl], 0.0)
            return carry2

        lax.fori_loop(0, CH, row, 0)
        pltpu.sync_copy(b0_v, w1_hbm.at[pl.ds(base, CH)])
        pltpu.sync_copy(b0_v, agg_sh.at[is_v], add=True)
        return carry

    lax.fori_loop(0, NCHUNK, chunk, 0)
    plsc.subcore_barrier()
    pltpu.sync_copy(agg_sh.at[pl.ds(sid * NPT, NPT)],
                    agg_hbm.at[cid, pl.ds(sid * NPT, NPT)])


_K7_KW = dict(
    out_type=jax.ShapeDtypeStruct((E,), _f32),
    mesh=_mesh,
    scratch_types=[
        pltpu.VMEM((CH,), jnp.int32),
        pltpu.VMEM((CH,), jnp.int32),
        pltpu.VMEM((CH, H), _f32),
        pltpu.VMEM((CH, H), _f32),
        pltpu.VMEM((CH, H), _f32),
        pltpu.VMEM((H, 16), _f32),
        pltpu.VMEM((16,), _f32),
        pltpu.VMEM((CH,), _f32),
        pltpu.SemaphoreType.DMA,
    ],
    compiler_params=pltpu.CompilerParams(use_tc_tiling_on_sc=False,
                                         needs_layout_passes=False),
)


def _k7_body(y3_hbm, y4_hbm, c1_hbm, src_hbm, dst_hbm, wc_hbm, bc_hbm,
                 sx_hbm, is_v, id_v, y3_v, y4_v, c1_v, wc_v, bc_v, out_v, sem):
    wid = lax.axis_index("s") * NC + lax.axis_index("c")
    pltpu.sync_copy(wc_hbm, wc_v)
    pltpu.sync_copy(bc_hbm, bc_v)

    def chunk(c, carry):
        base = pl.multiple_of(wid * EPW + c * CH, 8)
        pltpu.sync_copy(src_hbm.at[pl.ds(base, CH)], is_v)
        pltpu.sync_copy(dst_hbm.at[pl.ds(base, CH)], id_v)
        pltpu.async_copy(y3_hbm.at[is_v], y3_v, sem).wait()
        pltpu.async_copy(y4_hbm.at[id_v], y4_v, sem).wait()
        pltpu.sync_copy(c1_hbm.at[pl.ds(base, CH)], c1_v)

        for g in range(CH // 16):
            rows = lax.iota(jnp.int32, 16) + (g * 16)
            acc = jnp.zeros((16,), _f32)
            for f in range(H):
                cf = jnp.full((16,), f, jnp.int32)
                v = (plsc.load_gather(y3_v, [rows, cf])
                     + plsc.load_gather(y4_v, [rows, cf])
                     + plsc.load_gather(c1_v, [rows, cf]))
                v = jnp.maximum(v, 0.0)
                acc = acc + v * wc_v[f, pl.ds(0, 16)]
            z = acc + bc_v[...]
            out_v[pl.ds(g * 16, 16)] = 1.0 / (1.0 + jnp.exp(-z))

        pltpu.sync_copy(out_v, sx_hbm.at[pl.ds(base, CH)])
        return carry

    lax.fori_loop(0, NCHUNK, chunk, 0)


_k7_classify = pl.kernel(_k7_body, **_K7_KW)


BE = 3200


def _k2_body(w_ref, wt1_ref, wt2_ref, b2_ref, a_ref, b_ref):
    w = w_ref[...]
    a_ref[...] = jnp.dot(w, wt1_ref[...], preferred_element_type=_f32)
    b_ref[...] = jnp.dot(w, wt2_ref[...], preferred_element_type=_f32) + b2_ref[...]


def _k2_project(w0, wp1t, wse1t, bB0):
    grid = E // BE
    return pl.pallas_call(
        _k2_body,
        grid=(grid,),
        in_specs=[
            pl.BlockSpec((BE, DF), lambda i: (i, 0)),
            pl.BlockSpec((DF, H), lambda i: (0, 0)),
            pl.BlockSpec((DF, H), lambda i: (0, 0)),
            pl.BlockSpec((1, H), lambda i: (0, 0)),
        ],
        out_specs=[
            pl.BlockSpec((BE, H), lambda i: (i, 0)),
            pl.BlockSpec((BE, H), lambda i: (i, 0)),
        ],
        out_shape=[
            jax.ShapeDtypeStruct((E, H), _f32),
            jax.ShapeDtypeStruct((E, H), _f32),
        ],
    )(w0, wp1t, wse1t, bB0)


def _k4_body(aggp_ref, dinv_ref, b1_ref, m1_ref, m2_ref,
             xn_ref, y1_ref, y2_ref):
    agg = aggp_ref[0, :N] + aggp_ref[1, :N]
    xn = jnp.maximum(agg * dinv_ref[...] + b1_ref[...], 0.0)
    xn_ref[...] = xn
    y1_ref[...] = jnp.dot(xn, m1_ref[...], preferred_element_type=_f32)
    y2_ref[...] = jnp.dot(xn, m2_ref[...], preferred_element_type=_f32)


def _k4_node1(aggP, dinv, b1, m1, m2):
    return pl.pallas_call(
        _k4_body,
        out_shape=[
            jax.ShapeDtypeStruct((N, H), _f32),
            jax.ShapeDtypeStruct((N, H), _f32),
            jax.ShapeDtypeStruct((N, H), _f32),
        ],
    )(aggP, dinv, b1, m1, m2)


def _k6a_body(aggp_ref, dinv_ref, xn_ref, wp2t_ref, ws2t_ref, b2_ref,
              m3_ref, m4_ref, y3_ref, y4_ref):
    agg = (aggp_ref[0, :N] + aggp_ref[1, :N]) * dinv_ref[...]
    xn2 = jnp.maximum(
        jnp.dot(agg, wp2t_ref[...], preferred_element_type=_f32)
        + jnp.dot(xn_ref[...], ws2t_ref[...], preferred_element_type=_f32)
        + b2_ref[...], 0.0)
    y3_ref[...] = jnp.dot(xn2, m3_ref[...], preferred_element_type=_f32)
    y4_ref[...] = jnp.dot(xn2, m4_ref[...], preferred_element_type=_f32)


def _k6a_node2(agg2P, dinv, xn, wp2t, ws2t, b2, m3, m4):
    return pl.pallas_call(
        _k6a_body,
        out_shape=[
            jax.ShapeDtypeStruct((N, H), _f32),
            jax.ShapeDtypeStruct((N, H), _f32),
        ],
    )(agg2P, dinv, xn, wp2t, ws2t, b2, m3, m4)


def _k6b_body(w_ref, wt_ref, b_ref, o_ref):
    o_ref[...] = (jnp.dot(w_ref[...], wt_ref[...], preferred_element_type=_f32)
                  + b_ref[...])


def _k6b_project(w1, wse2t, bC1):
    grid = E // BE
    return pl.pallas_call(
        _k6b_body,
        grid=(grid,),
        in_specs=[
            pl.BlockSpec((BE, H), lambda i: (i, 0)),
            pl.BlockSpec((H, H), lambda i: (0, 0)),
            pl.BlockSpec((1, H), lambda i: (0, 0)),
        ],
        out_specs=pl.BlockSpec((BE, H), lambda i: (i, 0)),
        out_shape=jax.ShapeDtypeStruct((E, H), _f32),
    )(w1, wse2t, bC1)


def kernel(X, edge_index, D, Wp1, bp1, Ws1, bs1, Wpe1, bpe1, Wse1, bse1,
           Wp2, bp2, Ws2, bs2, Wpe2, bpe2, Wse2, bse2, Wc, bc):
    src = edge_index[0]
    dst = edge_index[1]

    wp1t = Wp1.T
    wse1t = Wse1.T
    bB0 = (bpe1 + bse1)[None, :]
    wa1, wb1 = Wpe1[:, :H], Wpe1[:, H:]
    m1 = ((wa1 + wb1) / 2).T
    m2 = ((wb1 - wa1) / 2).T
    wa2, wb2 = Wpe2[:, :H], Wpe2[:, H:]
    m3 = ((wa2 + wb2) / 2).T
    m4 = ((wb2 - wa2) / 2).T
    b1 = (bp1 + bs1)[None, :]
    b2 = (bp2 + bs2)[None, :]
    bC1 = (bpe2 + bse2)[None, :]
    wp2t = Wp2.T
    ws2t = Ws2.T
    wse2t = Wse2.T
    dinv = (1.0 / D)[:, None]
    wcb = jnp.broadcast_to(Wc[0][:, None], (H, 16))
    bc16 = jnp.broadcast_to(bc, (16,))

    w0 = _k1_absdiff(X, src, dst)
    a0, b0 = _k2_project(w0, wp1t, wse1t, bB0)
    aggP = _k3_segsum(a0, src)
    xn, y1, y2 = _k4_node1(aggP, dinv, b1, m1, m2)
    w1, agg2P = _k5_edge1(y1, y2, b0, src, dst)
    y3, y4 = _k6a_node2(agg2P, dinv, xn, wp2t, ws2t, b2, m3, m4)
    c1 = _k6b_project(w1, wse2t, bC1)
    sx = _k7_classify(y3, y4, c1, src, dst, wcb, bc16)
    return sx


# speedup vs baseline: 1.9642x; 1.0497x over previous
import functools

import jax
import jax.numpy as jnp
from jax import lax
from jax.experimental import pallas as pl
from jax.experimental.pallas import tpu as pltpu
from jax.experimental.pallas import tpu_sc as plsc

N = 10000
E = 320000
DF = 128
H = 32

NC = 2
NS = 16
NW = NC * NS
EPW = E // NW
CH = 80
NCHUNK = EPW // CH
NPAD = 10240
NPT = NPAD // NS

_mesh = plsc.VectorSubcoreMesh(core_axis_name="c", subcore_axis_name="s")

_f32 = jnp.float32


@functools.partial(
    pl.kernel,
    out_type=jax.ShapeDtypeStruct((E, DF), _f32),
    mesh=_mesh,
    scratch_types=[
        pltpu.VMEM((CH,), jnp.int32),
        pltpu.VMEM((CH,), jnp.int32),
        pltpu.VMEM((CH, DF), _f32),
        pltpu.VMEM((CH, DF), _f32),
        pltpu.SemaphoreType.DMA,
    ],
)
def _k1_absdiff(x_hbm, src_hbm, dst_hbm, w0_hbm, is_v, id_v, xs_v, xd_v, sem):
    wid = lax.axis_index("s") * NC + lax.axis_index("c")

    def chunk(c, carry):
        base = pl.multiple_of(wid * EPW + c * CH, 8)
        pltpu.sync_copy(src_hbm.at[pl.ds(base, CH)], is_v)
        pltpu.sync_copy(dst_hbm.at[pl.ds(base, CH)], id_v)
        pltpu.async_copy(x_hbm.at[is_v], xs_v, sem).wait()
        pltpu.async_copy(x_hbm.at[id_v], xd_v, sem).wait()

        def row(r, carry2):
            for j in range(DF // 16):
                sl = pl.ds(j * 16, 16)
                xs_v[r, sl] = jnp.abs(xs_v[r, sl] - xd_v[r, sl])
            return carry2

        lax.fori_loop(0, CH, row, 0)
        pltpu.sync_copy(xs_v, w0_hbm.at[pl.ds(base, CH)])
        return carry

    lax.fori_loop(0, NCHUNK, chunk, 0)


@functools.partial(
    pl.kernel,
    out_type=jax.ShapeDtypeStruct((NC, NPAD, H), _f32),
    mesh=_mesh,
    scratch_types=[
        pltpu.VMEM((CH,), jnp.int32),
        pltpu.VMEM((CH, H), _f32),
        pltpu.VMEM((NPT, H), _f32),
        pltpu.VMEM_SHARED((NPAD, H), _f32),
        pltpu.SemaphoreType.DMA,
    ],
    compiler_params=pltpu.CompilerParams(use_tc_tiling_on_sc=False),
)
def _k3_segsum(a0_hbm, src_hbm, out_hbm, idx_v, val_v, z_v, agg_sh, sem):
    cid = lax.axis_index("c")
    sid = lax.axis_index("s")
    wid = sid * NC + cid

    def zrow(r, carry):
        for j in range(H // 16):
            z_v[r, pl.ds(j * 16, 16)] = jnp.zeros((16,), _f32)
        return carry

    lax.fori_loop(0, NPT, zrow, 0)
    pltpu.sync_copy(z_v, agg_sh.at[pl.ds(sid * NPT, NPT)])
    plsc.subcore_barrier()

    def chunk(c, carry):
        base = pl.multiple_of(wid * EPW + c * CH, 8)
        pltpu.sync_copy(src_hbm.at[pl.ds(base, CH)], idx_v)
        pltpu.sync_copy(a0_hbm.at[pl.ds(base, CH)], val_v)
        pltpu.sync_copy(val_v, agg_sh.at[idx_v], add=True)
        return carry

    lax.fori_loop(0, NCHUNK, chunk, 0)
    plsc.subcore_barrier()
    pltpu.sync_copy(agg_sh.at[pl.ds(sid * NPT, NPT)], z_v)
    pltpu.sync_copy(z_v, out_hbm.at[cid, pl.ds(sid * NPT, NPT)])


@functools.partial(
    pl.kernel,
    out_type=[
        jax.ShapeDtypeStruct((E, H), _f32),
        jax.ShapeDtypeStruct((NC, NPAD, H), _f32),
    ],
    mesh=_mesh,
    scratch_types=[
        pltpu.VMEM((CH,), jnp.int32),
        pltpu.VMEM((CH,), jnp.int32),
        pltpu.VMEM((CH, H), _f32),
        pltpu.VMEM((CH, H), _f32),
        pltpu.VMEM((CH, H), _f32),
        pltpu.VMEM((NPT, H), _f32),
        pltpu.VMEM_SHARED((NPAD, H), _f32),
        pltpu.SemaphoreType.DMA,
    ],
    compiler_params=pltpu.CompilerParams(use_tc_tiling_on_sc=False),
)
def _k5_edge1(y1_hbm, y2_hbm, b0_hbm, src_hbm, dst_hbm, w1_hbm, agg_hbm,
              is_v, id_v, y1_v, y2_v, b0_v, z_v, agg_sh, sem):
    cid = lax.axis_index("c")
    sid = lax.axis_index("s")
    wid = sid * NC + cid

    def zrow(r, carry):
        for j in range(H // 16):
            z_v[r, pl.ds(j * 16, 16)] = jnp.zeros((16,), _f32)
        return carry

    lax.fori_loop(0, NPT, zrow, 0)
    pltpu.sync_copy(z_v, agg_sh.at[pl.ds(sid * NPT, NPT)])
    plsc.subcore_barrier()

    def chunk(c, carry):
        base = pl.multiple_of(wid * EPW + c * CH, 8)
        pltpu.sync_copy(src_hbm.at[pl.ds(base, CH)], is_v)
        pltpu.sync_copy(dst_hbm.at[pl.ds(base, CH)], id_v)
        pltpu.async_copy(y1_hbm.at[is_v], y1_v, sem).wait()
        pltpu.async_copy(y2_hbm.at[id_v], y2_v, sem).wait()
        pltpu.sync_copy(b0_hbm.at[pl.ds(base, CH)], b0_v)

        def row(r, carry2):
            for j in range(H // 16):
                sl = pl.ds(j * 16, 16)
                b0_v[r, sl] = jnp.maximum(
                    y1_v[r, sl] + y2_v[r, sl] + b0_v[r, sl], 0.0)
            return carry2

        lax.fori_loop(0, CH, row, 0)
        pltpu.sync_copy(b0_v, w1_hbm.at[pl.ds(base, CH)])
        pltpu.sync_copy(b0_v, agg_sh.at[is_v], add=True)
        return carry

    lax.fori_loop(0, NCHUNK, chunk, 0)
    plsc.subcore_barrier()
    pltpu.sync_copy(agg_sh.at[pl.ds(sid * NPT, NPT)],
                    agg_hbm.at[cid, pl.ds(sid * NPT, NPT)])


_K7_KW = dict(
    out_type=jax.ShapeDtypeStruct((E, H), _f32),
    mesh=_mesh,
    scratch_types=[
        pltpu.VMEM((CH,), jnp.int32),
        pltpu.VMEM((CH,), jnp.int32),
        pltpu.VMEM((CH, H), _f32),
        pltpu.VMEM((CH, H), _f32),
        pltpu.VMEM((CH, H), _f32),
        pltpu.SemaphoreType.DMA,
    ],
    compiler_params=pltpu.CompilerParams(use_tc_tiling_on_sc=False),
)


def _k7_body(y3_hbm, y4_hbm, c1_hbm, src_hbm, dst_hbm,
             v_hbm, is_v, id_v, y3_v, y4_v, c1_v, sem):
    wid = lax.axis_index("s") * NC + lax.axis_index("c")

    def chunk(c, carry):
        base = pl.multiple_of(wid * EPW + c * CH, 8)
        pltpu.sync_copy(src_hbm.at[pl.ds(base, CH)], is_v)
        pltpu.sync_copy(dst_hbm.at[pl.ds(base, CH)], id_v)
        pltpu.async_copy(y3_hbm.at[is_v], y3_v, sem).wait()
        pltpu.async_copy(y4_hbm.at[id_v], y4_v, sem).wait()
        pltpu.sync_copy(c1_hbm.at[pl.ds(base, CH)], c1_v)

        def row(r, carry2):
            for j in range(H // 16):
                sl = pl.ds(j * 16, 16)
                c1_v[r, sl] = y3_v[r, sl] + y4_v[r, sl] + c1_v[r, sl]
            return carry2

        lax.fori_loop(0, CH, row, 0)
        pltpu.sync_copy(c1_v, v_hbm.at[pl.ds(base, CH)])
        return carry

    lax.fori_loop(0, NCHUNK, chunk, 0)


_k7_combine = pl.kernel(_k7_body, **_K7_KW)


def _k8_body(v_ref, wc_ref, bc_ref, o_ref):
    w2 = jnp.maximum(v_ref[...], 0.0)
    z = jnp.dot(w2, wc_ref[...], preferred_element_type=_f32) + bc_ref[...]
    o_ref[...] = 1.0 / (1.0 + jnp.exp(-z))


def _k8_classify(v, wc_col, bc11):
    grid = E // BE
    return pl.pallas_call(
        _k8_body,
        grid=(grid,),
        in_specs=[
            pl.BlockSpec((BE, H), lambda i: (i, 0)),
            pl.BlockSpec((H, 1), lambda i: (0, 0)),
            pl.BlockSpec((1, 1), lambda i: (0, 0)),
        ],
        out_specs=pl.BlockSpec((BE, 1), lambda i: (i, 0)),
        out_shape=jax.ShapeDtypeStruct((E, 1), _f32),
    )(v, wc_col, bc11)


BE = 3200


def _k2_body(w_ref, wt1_ref, wt2_ref, b2_ref, a_ref, b_ref):
    w = w_ref[...]
    a_ref[...] = jnp.dot(w, wt1_ref[...], preferred_element_type=_f32)
    b_ref[...] = jnp.dot(w, wt2_ref[...], preferred_element_type=_f32) + b2_ref[...]


def _k2_project(w0, wp1t, wse1t, bB0):
    grid = E // BE
    return pl.pallas_call(
        _k2_body,
        grid=(grid,),
        in_specs=[
            pl.BlockSpec((BE, DF), lambda i: (i, 0)),
            pl.BlockSpec((DF, H), lambda i: (0, 0)),
            pl.BlockSpec((DF, H), lambda i: (0, 0)),
            pl.BlockSpec((1, H), lambda i: (0, 0)),
        ],
        out_specs=[
            pl.BlockSpec((BE, H), lambda i: (i, 0)),
            pl.BlockSpec((BE, H), lambda i: (i, 0)),
        ],
        out_shape=[
            jax.ShapeDtypeStruct((E, H), _f32),
            jax.ShapeDtypeStruct((E, H), _f32),
        ],
    )(w0, wp1t, wse1t, bB0)


def _k4_body(aggp_ref, dinv_ref, b1_ref, m1_ref, m2_ref,
             xn_ref, y1_ref, y2_ref):
    agg = aggp_ref[0, :N] + aggp_ref[1, :N]
    xn = jnp.maximum(agg * dinv_ref[...] + b1_ref[...], 0.0)
    xn_ref[...] = xn
    y1_ref[...] = jnp.dot(xn, m1_ref[...], preferred_element_type=_f32)
    y2_ref[...] = jnp.dot(xn, m2_ref[...], preferred_element_type=_f32)


def _k4_node1(aggP, dinv, b1, m1, m2):
    return pl.pallas_call(
        _k4_body,
        out_shape=[
            jax.ShapeDtypeStruct((N, H), _f32),
            jax.ShapeDtypeStruct((N, H), _f32),
            jax.ShapeDtypeStruct((N, H), _f32),
        ],
    )(aggP, dinv, b1, m1, m2)


def _k6a_body(aggp_ref, dinv_ref, xn_ref, wp2t_ref, ws2t_ref, b2_ref,
              m3_ref, m4_ref, y3_ref, y4_ref):
    agg = (aggp_ref[0, :N] + aggp_ref[1, :N]) * dinv_ref[...]
    xn2 = jnp.maximum(
        jnp.dot(agg, wp2t_ref[...], preferred_element_type=_f32)
        + jnp.dot(xn_ref[...], ws2t_ref[...], preferred_element_type=_f32)
        + b2_ref[...], 0.0)
    y3_ref[...] = jnp.dot(xn2, m3_ref[...], preferred_element_type=_f32)
    y4_ref[...] = jnp.dot(xn2, m4_ref[...], preferred_element_type=_f32)


def _k6a_node2(agg2P, dinv, xn, wp2t, ws2t, b2, m3, m4):
    return pl.pallas_call(
        _k6a_body,
        out_shape=[
            jax.ShapeDtypeStruct((N, H), _f32),
            jax.ShapeDtypeStruct((N, H), _f32),
        ],
    )(agg2P, dinv, xn, wp2t, ws2t, b2, m3, m4)


def _k6b_body(w_ref, wt_ref, b_ref, o_ref):
    o_ref[...] = (jnp.dot(w_ref[...], wt_ref[...], preferred_element_type=_f32)
                  + b_ref[...])


def _k6b_project(w1, wse2t, bC1):
    grid = E // BE
    return pl.pallas_call(
        _k6b_body,
        grid=(grid,),
        in_specs=[
            pl.BlockSpec((BE, H), lambda i: (i, 0)),
            pl.BlockSpec((H, H), lambda i: (0, 0)),
            pl.BlockSpec((1, H), lambda i: (0, 0)),
        ],
        out_specs=pl.BlockSpec((BE, H), lambda i: (i, 0)),
        out_shape=jax.ShapeDtypeStruct((E, H), _f32),
    )(w1, wse2t, bC1)


def kernel(X, edge_index, D, Wp1, bp1, Ws1, bs1, Wpe1, bpe1, Wse1, bse1,
           Wp2, bp2, Ws2, bs2, Wpe2, bpe2, Wse2, bse2, Wc, bc):
    src = edge_index[0]
    dst = edge_index[1]

    wp1t = Wp1.T
    wse1t = Wse1.T
    bB0 = (bpe1 + bse1)[None, :]
    wa1, wb1 = Wpe1[:, :H], Wpe1[:, H:]
    m1 = ((wa1 + wb1) / 2).T
    m2 = ((wb1 - wa1) / 2).T
    wa2, wb2 = Wpe2[:, :H], Wpe2[:, H:]
    m3 = ((wa2 + wb2) / 2).T
    m4 = ((wb2 - wa2) / 2).T
    b1 = (bp1 + bs1)[None, :]
    b2 = (bp2 + bs2)[None, :]
    bC1 = (bpe2 + bse2)[None, :]
    wp2t = Wp2.T
    ws2t = Ws2.T
    wse2t = Wse2.T
    dinv = (1.0 / D)[:, None]
    wc_col = Wc.T
    bc11 = bc[None, :]

    w0 = _k1_absdiff(X, src, dst)
    a0, b0 = _k2_project(w0, wp1t, wse1t, bB0)
    aggP = _k3_segsum(a0, src)
    xn, y1, y2 = _k4_node1(aggP, dinv, b1, m1, m2)
    w1, agg2P = _k5_edge1(y1, y2, b0, src, dst)
    y3, y4 = _k6a_node2(agg2P, dinv, xn, wp2t, ws2t, b2, m3, m4)
    c1 = _k6b_project(w1, wse2t, bC1)
    v = _k7_combine(y3, y4, c1, src, dst)
    sx = _k8_classify(v, wc_col, bc11)
    return sx[:, 0]


# speedup vs baseline: 2.4833x; 1.2643x over previous
import functools

import jax
import jax.numpy as jnp
from jax import lax
from jax.experimental import pallas as pl
from jax.experimental.pallas import tpu as pltpu
from jax.experimental.pallas import tpu_sc as plsc

N = 10000
E = 320000
DF = 128
H = 32

NC = 2
NS = 16
NW = NC * NS
EPW = E // NW
CH = 80
NCHUNK = EPW // CH
NPAD = 10240
NPT = NPAD // NS

_mesh = plsc.VectorSubcoreMesh(core_axis_name="c", subcore_axis_name="s")

_f32 = jnp.float32


@functools.partial(
    pl.kernel,
    out_type=jax.ShapeDtypeStruct((E, DF), _f32),
    mesh=_mesh,
    scratch_types=[
        pltpu.VMEM((CH,), jnp.int32),
        pltpu.VMEM((CH,), jnp.int32),
        pltpu.VMEM((CH, DF), _f32),
        pltpu.VMEM((CH, DF), _f32),
        pltpu.SemaphoreType.DMA,
        pltpu.SemaphoreType.DMA,
    ],
)
def _k1_absdiff(x_hbm, src_hbm, dst_hbm, w0_hbm, is_v, id_v, xs_v, xd_v, sem, sem2):
    wid = lax.axis_index("s") * NC + lax.axis_index("c")

    def chunk(c, carry):
        base = pl.multiple_of(wid * EPW + c * CH, 8)
        la = pltpu.async_copy(src_hbm.at[pl.ds(base, CH)], is_v, sem2)
        lb = pltpu.async_copy(dst_hbm.at[pl.ds(base, CH)], id_v, sem2)
        la.wait()
        lb.wait()
        ga = pltpu.async_copy(x_hbm.at[is_v], xs_v, sem)
        gb = pltpu.async_copy(x_hbm.at[id_v], xd_v, sem)
        ga.wait()
        gb.wait()

        def row(r, carry2):
            for j in range(DF // 16):
                sl = pl.ds(j * 16, 16)
                xs_v[r, sl] = jnp.abs(xs_v[r, sl] - xd_v[r, sl])
            return carry2

        lax.fori_loop(0, CH, row, 0)
        pltpu.sync_copy(xs_v, w0_hbm.at[pl.ds(base, CH)])
        return carry

    lax.fori_loop(0, NCHUNK, chunk, 0)


@functools.partial(
    pl.kernel,
    out_type=jax.ShapeDtypeStruct((NC, NPAD, H), _f32),
    mesh=_mesh,
    scratch_types=[
        pltpu.VMEM((CH,), jnp.int32),
        pltpu.VMEM((CH, H), _f32),
        pltpu.VMEM((NPT, H), _f32),
        pltpu.VMEM_SHARED((NPAD, H), _f32),
        pltpu.SemaphoreType.DMA,
    ],
    compiler_params=pltpu.CompilerParams(use_tc_tiling_on_sc=False),
)
def _k3_segsum(a0_hbm, ei_hbm, out_hbm, idx_v, val_v, z_v, agg_sh, sem):
    cid = lax.axis_index("c")
    sid = lax.axis_index("s")
    wid = sid * NC + cid

    def zrow(r, carry):
        for j in range(H // 16):
            z_v[r, pl.ds(j * 16, 16)] = jnp.zeros((16,), _f32)
        return carry

    lax.fori_loop(0, NPT, zrow, 0)
    pltpu.sync_copy(z_v, agg_sh.at[pl.ds(sid * NPT, NPT)])
    plsc.subcore_barrier()

    def chunk(c, carry):
        base = pl.multiple_of(wid * EPW + c * CH, 8)
        la = pltpu.async_copy(ei_hbm.at[0, pl.ds(base, CH)], idx_v, sem)
        lb = pltpu.async_copy(a0_hbm.at[pl.ds(base, CH)], val_v, sem)
        la.wait()
        lb.wait()
        pltpu.sync_copy(val_v, agg_sh.at[idx_v], add=True)
        return carry

    lax.fori_loop(0, NCHUNK, chunk, 0)
    plsc.subcore_barrier()
    pltpu.sync_copy(agg_sh.at[pl.ds(sid * NPT, NPT)], z_v)
    pltpu.sync_copy(z_v, out_hbm.at[cid, pl.ds(sid * NPT, NPT)])


@functools.partial(
    pl.kernel,
    out_type=[
        jax.ShapeDtypeStruct((E, H), _f32),
        jax.ShapeDtypeStruct((NC, NPAD, H), _f32),
    ],
    mesh=_mesh,
    scratch_types=[
        pltpu.VMEM((2, CH), jnp.int32),
        pltpu.VMEM((CH, H), _f32),
        pltpu.VMEM((CH, H), _f32),
        pltpu.VMEM((CH, H), _f32),
        pltpu.VMEM((NPT, H), _f32),
        pltpu.VMEM_SHARED((NPAD, H), _f32),
        pltpu.SemaphoreType.DMA,
        pltpu.SemaphoreType.DMA,
    ],
    compiler_params=pltpu.CompilerParams(use_tc_tiling_on_sc=False),
)
def _k5_edge1(y1_hbm, y2_hbm, b0_hbm, ei_hbm, w1_hbm, agg_hbm,
              idx_v, y1_v, y2_v, b0_v, z_v, agg_sh, sem, sem2):
    cid = lax.axis_index("c")
    sid = lax.axis_index("s")
    wid = sid * NC + cid

    def zrow(r, carry):
        for j in range(H // 16):
            z_v[r, pl.ds(j * 16, 16)] = jnp.zeros((16,), _f32)
        return carry

    lax.fori_loop(0, NPT, zrow, 0)
    pltpu.sync_copy(z_v, agg_sh.at[pl.ds(sid * NPT, NPT)])
    plsc.subcore_barrier()

    def chunk(c, carry):
        base = pl.multiple_of(wid * EPW + c * CH, 8)
        pltpu.sync_copy(ei_hbm.at[:, pl.ds(base, CH)], idx_v)
        lb = pltpu.async_copy(b0_hbm.at[pl.ds(base, CH)], b0_v, sem2)
        ga = pltpu.async_copy(y1_hbm.at[idx_v.at[0]], y1_v, sem)
        gb = pltpu.async_copy(y2_hbm.at[idx_v.at[1]], y2_v, sem)
        ga.wait()
        gb.wait()
        lb.wait()

        def row(r, carry2):
            for j in range(H // 16):
                sl = pl.ds(j * 16, 16)
                b0_v[r, sl] = jnp.maximum(
                    y1_v[r, sl] + y2_v[r, sl] + b0_v[r, sl], 0.0)
            return carry2

        lax.fori_loop(0, CH, row, 0)
        pltpu.sync_copy(b0_v, w1_hbm.at[pl.ds(base, CH)])
        pltpu.sync_copy(b0_v, agg_sh.at[idx_v.at[0]], add=True)
        return carry

    lax.fori_loop(0, NCHUNK, chunk, 0)
    plsc.subcore_barrier()
    pltpu.sync_copy(agg_sh.at[pl.ds(sid * NPT, NPT)],
                    agg_hbm.at[cid, pl.ds(sid * NPT, NPT)])


_K7_KW = dict(
    out_type=jax.ShapeDtypeStruct((E, H), _f32),
    mesh=_mesh,
    scratch_types=[
        pltpu.VMEM((2, CH), jnp.int32),
        pltpu.VMEM((CH, H), _f32),
        pltpu.VMEM((CH, H), _f32),
        pltpu.VMEM((CH, H), _f32),
        pltpu.SemaphoreType.DMA,
        pltpu.SemaphoreType.DMA,
    ],
    compiler_params=pltpu.CompilerParams(use_tc_tiling_on_sc=False),
)


def _k7_body(y3_hbm, y4_hbm, c1_hbm, ei_hbm,
             v_hbm, idx_v, y3_v, y4_v, c1_v, sem, sem2):
    wid = lax.axis_index("s") * NC + lax.axis_index("c")

    def chunk(c, carry):
        base = pl.multiple_of(wid * EPW + c * CH, 8)
        pltpu.sync_copy(ei_hbm.at[:, pl.ds(base, CH)], idx_v)
        lb = pltpu.async_copy(c1_hbm.at[pl.ds(base, CH)], c1_v, sem2)
        ga = pltpu.async_copy(y3_hbm.at[idx_v.at[0]], y3_v, sem)
        gb = pltpu.async_copy(y4_hbm.at[idx_v.at[1]], y4_v, sem)
        ga.wait()
        gb.wait()
        lb.wait()

        def row(r, carry2):
            for j in range(H // 16):
                sl = pl.ds(j * 16, 16)
                c1_v[r, sl] = y3_v[r, sl] + y4_v[r, sl] + c1_v[r, sl]
            return carry2

        lax.fori_loop(0, CH, row, 0)
        pltpu.sync_copy(c1_v, v_hbm.at[pl.ds(base, CH)])
        return carry

    lax.fori_loop(0, NCHUNK, chunk, 0)


_k7_combine = pl.kernel(_k7_body, **_K7_KW)


def _k8_body(v_ref, wc_ref, bc_ref, o_ref):
    w2 = jnp.maximum(v_ref[...], 0.0)
    z = jnp.dot(w2, wc_ref[...], preferred_element_type=_f32) + bc_ref[...]
    o_ref[...] = 1.0 / (1.0 + jnp.exp(-z))


def _k8_classify(v, wc_col, bc11):
    grid = E // BE
    return pl.pallas_call(
        _k8_body,
        grid=(grid,),
        in_specs=[
            pl.BlockSpec((BE, H), lambda i: (i, 0)),
            pl.BlockSpec((H, 1), lambda i: (0, 0)),
            pl.BlockSpec((1, 1), lambda i: (0, 0)),
        ],
        out_specs=pl.BlockSpec((BE, 1), lambda i: (i, 0)),
        out_shape=jax.ShapeDtypeStruct((E, 1), _f32),
    )(v, wc_col, bc11)


BE = 3200


def _k2_body(w_ref, wt1_ref, wt2_ref, b2_ref, a_ref, b_ref):
    w = w_ref[...]
    a_ref[...] = jnp.dot(w, wt1_ref[...], preferred_element_type=_f32)
    b_ref[...] = jnp.dot(w, wt2_ref[...], preferred_element_type=_f32) + b2_ref[...]


def _k2_project(w0, wp1t, wse1t, bB0):
    grid = E // BE
    return pl.pallas_call(
        _k2_body,
        grid=(grid,),
        in_specs=[
            pl.BlockSpec((BE, DF), lambda i: (i, 0)),
            pl.BlockSpec((DF, H), lambda i: (0, 0)),
            pl.BlockSpec((DF, H), lambda i: (0, 0)),
            pl.BlockSpec((1, H), lambda i: (0, 0)),
        ],
        out_specs=[
            pl.BlockSpec((BE, H), lambda i: (i, 0)),
            pl.BlockSpec((BE, H), lambda i: (i, 0)),
        ],
        out_shape=[
            jax.ShapeDtypeStruct((E, H), _f32),
            jax.ShapeDtypeStruct((E, H), _f32),
        ],
    )(w0, wp1t, wse1t, bB0)


def _k4_body(aggp_ref, dinv_ref, b1_ref, m1_ref, m2_ref,
             xn_ref, y1_ref, y2_ref):
    agg = aggp_ref[0, :N] + aggp_ref[1, :N]
    xn = jnp.maximum(agg * dinv_ref[...] + b1_ref[...], 0.0)
    xn_ref[...] = xn
    y1_ref[...] = jnp.dot(xn, m1_ref[...], preferred_element_type=_f32)
    y2_ref[...] = jnp.dot(xn, m2_ref[...], preferred_element_type=_f32)


def _k4_node1(aggP, dinv, b1, m1, m2):
    return pl.pallas_call(
        _k4_body,
        out_shape=[
            jax.ShapeDtypeStruct((N, H), _f32),
            jax.ShapeDtypeStruct((N, H), _f32),
            jax.ShapeDtypeStruct((N, H), _f32),
        ],
    )(aggP, dinv, b1, m1, m2)


def _k6a_body(aggp_ref, dinv_ref, xn_ref, wp2t_ref, ws2t_ref, b2_ref,
              m3_ref, m4_ref, y3_ref, y4_ref):
    agg = (aggp_ref[0, :N] + aggp_ref[1, :N]) * dinv_ref[...]
    xn2 = jnp.maximum(
        jnp.dot(agg, wp2t_ref[...], preferred_element_type=_f32)
        + jnp.dot(xn_ref[...], ws2t_ref[...], preferred_element_type=_f32)
        + b2_ref[...], 0.0)
    y3_ref[...] = jnp.dot(xn2, m3_ref[...], preferred_element_type=_f32)
    y4_ref[...] = jnp.dot(xn2, m4_ref[...], preferred_element_type=_f32)


def _k6a_node2(agg2P, dinv, xn, wp2t, ws2t, b2, m3, m4):
    return pl.pallas_call(
        _k6a_body,
        out_shape=[
            jax.ShapeDtypeStruct((N, H), _f32),
            jax.ShapeDtypeStruct((N, H), _f32),
        ],
    )(agg2P, dinv, xn, wp2t, ws2t, b2, m3, m4)


def _k6b_body(w_ref, wt_ref, b_ref, o_ref):
    o_ref[...] = (jnp.dot(w_ref[...], wt_ref[...], preferred_element_type=_f32)
                  + b_ref[...])


def _k6b_project(w1, wse2t, bC1):
    grid = E // BE
    return pl.pallas_call(
        _k6b_body,
        grid=(grid,),
        in_specs=[
            pl.BlockSpec((BE, H), lambda i: (i, 0)),
            pl.BlockSpec((H, H), lambda i: (0, 0)),
            pl.BlockSpec((1, H), lambda i: (0, 0)),
        ],
        out_specs=pl.BlockSpec((BE, H), lambda i: (i, 0)),
        out_shape=jax.ShapeDtypeStruct((E, H), _f32),
    )(w1, wse2t, bC1)


def kernel(X, edge_index, D, Wp1, bp1, Ws1, bs1, Wpe1, bpe1, Wse1, bse1,
           Wp2, bp2, Ws2, bs2, Wpe2, bpe2, Wse2, bse2, Wc, bc):
    wp1t = Wp1.T
    wse1t = Wse1.T
    bB0 = (bpe1 + bse1)[None, :]
    wa1, wb1 = Wpe1[:, :H], Wpe1[:, H:]
    m1 = ((wa1 + wb1) / 2).T
    m2 = ((wb1 - wa1) / 2).T
    wa2, wb2 = Wpe2[:, :H], Wpe2[:, H:]
    m3 = ((wa2 + wb2) / 2).T
    m4 = ((wb2 - wa2) / 2).T
    b1 = (bp1 + bs1)[None, :]
    b2 = (bp2 + bs2)[None, :]
    bC1 = (bpe2 + bse2)[None, :]
    wp2t = Wp2.T
    ws2t = Ws2.T
    wse2t = Wse2.T
    dinv = (1.0 / D)[:, None]
    wc_col = Wc.T
    bc11 = bc[None, :]

    w0 = _k1_absdiff(X, edge_index[0], edge_index[1])
    a0, b0 = _k2_project(w0, wp1t, wse1t, bB0)
    aggP = _k3_segsum(a0, edge_index)
    xn, y1, y2 = _k4_node1(aggP, dinv, b1, m1, m2)
    w1, agg2P = _k5_edge1(y1, y2, b0, edge_index)
    y3, y4 = _k6a_node2(agg2P, dinv, xn, wp2t, ws2t, b2, m3, m4)
    c1 = _k6b_project(w1, wse2t, bC1)
    v = _k7_combine(y3, y4, c1, edge_index)
    sx = _k8_classify(v, wc_col, bc11)
    return sx[:, 0]
